# Initial kernel scaffold; baseline (speedup 1.0000x reference)
import jax
import jax.numpy as jnp
from jax.experimental import pallas as pl


def kernel(diagonal_elements, off_diagonal_elements):
    raise NotImplementedError("write your pallas kernel here")


# speedup vs baseline: 606.8230x; 606.8230x over previous
import functools
import jax
import jax.numpy as jnp
from jax import lax
from jax.experimental import pallas as pl
from jax.experimental.pallas import tpu as pltpu
from jax.experimental.pallas import tpu_sc as plsc

N = 4096
L = (N * N - N) // 2
NW = 32
RPW = N // NW
BASE = 48
BUFW = BASE + N + 48
NBUF = 8
AHEAD = 4


def _il(c):
    return 512 * c + 544


def _sc_body(off_hbm, diag_hbm, zeros_hbm, out_hbm,
             buf, diagbuf, sem_a, sem_b, sem_z):
    info = plsc.get_sparse_core_info()
    nc = info.num_cores
    wid = lax.axis_index("s") * nc + lax.axis_index("c")

    pltpu.sync_copy(diag_hbm, diagbuf)
    for s in range(NBUF):
        pltpu.make_async_copy(
            zeros_hbm.at[pl.ds(0, BUFW - BASE - 544)],
            buf.at[pl.ds(s * BUFW + BASE + 544, BUFW - BASE - 544)],
            sem_z,
        ).start()
    for s in range(NBUF):
        pltpu.make_async_copy(
            zeros_hbm.at[pl.ds(0, BUFW - BASE - 544)],
            buf.at[pl.ds(s * BUFW + BASE + 544, BUFW - BASE - 544)],
            sem_z,
        ).wait()

    def issue_a(t):
        i = wid + t * NW
        tri = (i * (i - 1)) // 2
        s0 = (tri >> 3) << 3
        cls = i >> 9
        slot = lax.rem(t, NBUF)
        for c in range(8):
            il = _il(c)

            @pl.when(cls == c)
            def _():
                s0c = pl.multiple_of(jnp.minimum(s0, L - il), 8)
                pad = tri - s0c
                pltpu.make_async_copy(
                    off_hbm.at[pl.ds(s0c, il)],
                    buf.at[pl.ds(pl.multiple_of(slot * BUFW + BASE - pad, 8), il)],
                    sem_a.at[slot],
                ).start()

    for u in range(AHEAD):
        issue_a(u)

    zvec = jnp.zeros((16,), jnp.float32)
    lanes = lax.broadcasted_iota(jnp.int32, (16,), 0)

    def step(t, carry):
        i = wid + t * NW
        tri = (i * (i - 1)) // 2
        cls = i >> 9
        slot = lax.rem(t, NBUF)

        for c in range(8):
            il = _il(c)

            @pl.when(cls == c)
            def _():
                pltpu.make_async_copy(
                    off_hbm.at[pl.ds(0, il)],
                    buf.at[pl.ds(slot * BUFW, il)],
                    sem_a.at[slot],
                ).wait()

        p = BASE + i
        z0 = (p >> 4) << 4
        pos = p - z0
        dv16 = diagbuf[pl.ds((i >> 4) << 4, 16)]
        sb = slot * BUFW
        v = buf[pl.ds(sb + z0, 16)]
        v = jnp.where(lanes == pos, dv16,
                      jnp.where(lanes > pos, 0.0, v))
        buf[pl.ds(sb + z0, 16)] = v

        end = BASE + 544 + 512 * cls
        nz = (end - (z0 + 16) + 15) >> 4

        def zstep(u, _):
            buf[pl.ds(sb + z0 + 16 + u * 16, 16)] = zvec
            return 0

        lax.fori_loop(0, nz, zstep, 0)

        pltpu.make_async_copy(
            buf.at[pl.ds(sb + BASE, N)],
            out_hbm.at[pl.ds(pl.multiple_of(i * N, 8), N)],
            sem_b.at[slot],
        ).start()

        ta = t + AHEAD

        @pl.when(ta < RPW)
        def _():
            aslot = lax.rem(ta, NBUF)

            @pl.when(ta >= NBUF)
            def _():
                pltpu.make_async_copy(
                    buf.at[pl.ds(aslot * BUFW + BASE, N)],
                    out_hbm.at[pl.ds(0, N)],
                    sem_b.at[aslot],
                ).wait()

            issue_a(ta)

        return 0

    lax.fori_loop(0, RPW, step, 0)

    for s in range(NBUF):
        pltpu.make_async_copy(
            buf.at[pl.ds(s * BUFW + BASE, N)],
            out_hbm.at[pl.ds(0, N)],
            sem_b.at[s],
        ).wait()


def kernel(diagonal_elements, off_diagonal_elements):
    mesh = plsc.VectorSubcoreMesh(core_axis_name="c", subcore_axis_name="s")
    zeros_src = jnp.zeros((4096,), jnp.float32)
    run = pl.kernel(
        _sc_body,
        out_type=jax.ShapeDtypeStruct((N * N,), jnp.float32),
        mesh=mesh,
        scratch_types=[
            pltpu.VMEM((NBUF * BUFW,), jnp.float32),
            pltpu.VMEM((N,), jnp.float32),
            pltpu.SemaphoreType.DMA((NBUF,)),
            pltpu.SemaphoreType.DMA((NBUF,)),
            pltpu.SemaphoreType.DMA,
        ],
    )
    return run(off_diagonal_elements, diagonal_elements, zeros_src).reshape(N, N)


# speedup vs baseline: 611.4697x; 1.0077x over previous
import jax
import jax.numpy as jnp
from jax import lax
from jax.experimental import pallas as pl
from jax.experimental.pallas import tpu as pltpu
from jax.experimental.pallas import tpu_sc as plsc

N = 4096
L = (N * N - N) // 2
NW = 32
RPW = N // NW
BASE = 48
BUFW = BASE + N + 48
NBUF = 8
AHEAD = 4


NCLS = 16


def _il(c):
    return 256 * c + 280


def _sc_body(off_hbm, diag_hbm, zeros_hbm, out_hbm,
             buf, diagbuf, sem_a, sem_b, sem_z):
    info = plsc.get_sparse_core_info()
    nc = info.num_cores
    wid = lax.axis_index("s") * nc + lax.axis_index("c")

    pltpu.sync_copy(diag_hbm, diagbuf)
    for s in range(NBUF):
        pltpu.make_async_copy(
            zeros_hbm.at[pl.ds(0, BUFW - BASE - 280)],
            buf.at[pl.ds(s * BUFW + BASE + 280, BUFW - BASE - 280)],
            sem_z,
        ).start()
    for s in range(NBUF):
        pltpu.make_async_copy(
            zeros_hbm.at[pl.ds(0, BUFW - BASE - 280)],
            buf.at[pl.ds(s * BUFW + BASE + 280, BUFW - BASE - 280)],
            sem_z,
        ).wait()

    def issue_a(t):
        i = wid + t * NW
        tri = (i * (i - 1)) // 2
        s0 = (tri >> 3) << 3
        cls = i >> 8
        slot = lax.rem(t, NBUF)
        for c in range(NCLS):
            il = _il(c)

            @pl.when(cls == c)
            def _():
                s0c = pl.multiple_of(jnp.minimum(s0, L - il), 8)
                pad = tri - s0c
                pltpu.make_async_copy(
                    off_hbm.at[pl.ds(s0c, il)],
                    buf.at[pl.ds(pl.multiple_of(slot * BUFW + BASE - pad, 8), il)],
                    sem_a.at[slot],
                ).start()

    for u in range(AHEAD):
        issue_a(u)

    zvec = jnp.zeros((16,), jnp.float32)
    lanes = lax.broadcasted_iota(jnp.int32, (16,), 0)

    def step(t, carry):
        i = wid + t * NW
        cls = i >> 8
        slot = lax.rem(t, NBUF)

        for c in range(NCLS):
            il = _il(c)

            @pl.when(cls == c)
            def _():
                pltpu.make_async_copy(
                    off_hbm.at[pl.ds(0, il)],
                    buf.at[pl.ds(slot * BUFW, il)],
                    sem_a.at[slot],
                ).wait()

        p = BASE + i
        z0 = (p >> 4) << 4
        pos = p - z0
        dv16 = diagbuf[pl.ds((i >> 4) << 4, 16)]
        sb = slot * BUFW
        v = buf[pl.ds(sb + z0, 16)]
        v = jnp.where(lanes == pos, dv16,
                      jnp.where(lanes > pos, 0.0, v))
        buf[pl.ds(sb + z0, 16)] = v

        end = BASE + 280 + 256 * cls
        nz = (end - (z0 + 16) + 15) >> 4

        def zstep(u, _):
            buf[pl.ds(sb + z0 + 16 + u * 16, 16)] = zvec
            return 0

        lax.fori_loop(0, nz, zstep, 0)

        pltpu.make_async_copy(
            buf.at[pl.ds(sb + BASE, N)],
            out_hbm.at[pl.ds(pl.multiple_of(i * N, 8), N)],
            sem_b.at[slot],
        ).start()

        ta = t + AHEAD

        @pl.when(ta < RPW)
        def _():
            aslot = lax.rem(ta, NBUF)

            @pl.when(ta >= NBUF)
            def _():
                pltpu.make_async_copy(
                    buf.at[pl.ds(aslot * BUFW + BASE, N)],
                    out_hbm.at[pl.ds(0, N)],
                    sem_b.at[aslot],
                ).wait()

            issue_a(ta)

        return 0

    lax.fori_loop(0, RPW, step, 0)

    for s in range(NBUF):
        pltpu.make_async_copy(
            buf.at[pl.ds(s * BUFW + BASE, N)],
            out_hbm.at[pl.ds(0, N)],
            sem_b.at[s],
        ).wait()


def kernel(diagonal_elements, off_diagonal_elements):
    mesh = plsc.VectorSubcoreMesh(core_axis_name="c", subcore_axis_name="s")
    zeros_src = jnp.zeros((N,), jnp.float32)
    run = pl.kernel(
        _sc_body,
        out_type=jax.ShapeDtypeStruct((N * N,), jnp.float32),
        mesh=mesh,
        scratch_types=[
            pltpu.VMEM((NBUF * BUFW,), jnp.float32),
            pltpu.VMEM((N,), jnp.float32),
            pltpu.SemaphoreType.DMA((NBUF,)),
            pltpu.SemaphoreType.DMA((NBUF,)),
            pltpu.SemaphoreType.DMA,
        ],
    )
    return run(off_diagonal_elements, diagonal_elements, zeros_src).reshape(N, N)
